# baseline (speedup 1.0000x reference)
import functools

import jax
import jax.numpy as jnp
from jax import lax
from jax.experimental import pallas as pl
from jax.experimental.pallas import tpu as pltpu
from jax.experimental.pallas import tpu_sc as plsc

B = 2
C = 16
H = 512
W = 512
PADF = 20.0
NBAND = 4
BAND = H // NBAND
BANDW = BAND * W
ROWBLK = 8
BLKPIX = ROWBLK * W
NBLK = H // ROWBLK

_mesh = plsc.VectorSubcoreMesh(core_axis_name="c", subcore_axis_name="s")


@functools.partial(
    pl.kernel,
    out_type=jax.ShapeDtypeStruct((B * C, H * W), jnp.float32),
    mesh=_mesh,
    scratch_types=[
        pltpu.VMEM((BANDW,), jnp.float32),
        pltpu.VMEM((BLKPIX * 2,), jnp.float32),
        pltpu.VMEM((BLKPIX,), jnp.float32),
    ],
    compiler_params=pltpu.CompilerParams(needs_layout_passes=False),
)
def _sc_splat(im_hbm, flow_hbm, out_hbm, acc, flv, val):
    wid = lax.axis_index("s") * 2 + lax.axis_index("c")
    b = wid // C
    lanes = lax.iota(jnp.int32, 16)

    for band in range(NBAND):
        lo_row = band * BAND
        hi_row = lo_row + BAND
        lo = band * BANDW

        def zbody(i, _):
            base = i * 128
            for j in range(8):
                acc[pl.ds(base + j * 16, 16)] = jnp.zeros((16,), jnp.float32)
            return 0

        lax.fori_loop(0, BANDW // 128, zbody, 0)

        def blkbody(blk, _):
            pltpu.sync_copy(
                flow_hbm.at[b, pl.ds(blk * (BLKPIX * 2), BLKPIX * 2)], flv)
            pltpu.sync_copy(im_hbm.at[wid, pl.ds(blk * BLKPIX, BLKPIX)], val)

            def chunk(ci, _):
                off = ci * 16
                gp = blk * BLKPIX + off + lanes
                dy = plsc.load_gather(flv, [off * 2 + lanes * 2 + 1])
                row = lax.shift_right_logical(gp, 9)
                ty = row.astype(jnp.float32) + dy + PADF
                y0 = ty.astype(jnp.int32)
                y0 = jnp.where(y0.astype(jnp.float32) > ty, y0 - 1, y0)
                hitv = (y0 >= lo_row - 1) & (y0 < hi_row)

                @pl.when(jnp.any(hitv))
                def _():
                    dx = plsc.load_gather(flv, [off * 2 + lanes * 2])
                    col = jnp.bitwise_and(gp, W - 1)
                    tx = col.astype(jnp.float32) + dx + PADF
                    x0 = tx.astype(jnp.int32)
                    x0 = jnp.where(x0.astype(jnp.float32) > tx, x0 - 1, x0)
                    fx = tx - x0.astype(jnp.float32)
                    fy = ty - y0.astype(jnp.float32)
                    gx = 1.0 - fx
                    gy = 1.0 - fy
                    x1 = x0 + 1
                    y1 = y0 + 1
                    vx0 = (x0 >= 0) & (x0 < W)
                    vx1 = (x1 >= 0) & (x1 < W)
                    vy0 = (y0 >= lo_row) & (y0 < hi_row)
                    vy1 = (y1 >= lo_row) & (y1 < hi_row)
                    ly0 = y0 * W - lo
                    ly1 = y1 * W - lo
                    v = val[pl.ds(off, 16)]
                    for lyv, xv, mv, wv in (
                            (ly0, x0, vx0 & vy0, gx * gy),
                            (ly0, x1, vx1 & vy0, fx * gy),
                            (ly1, x0, vx0 & vy1, gx * fy),
                            (ly1, x1, vx1 & vy1, fx * fy),
                    ):
                        idx = jnp.where(mv, lyv + xv, 0)
                        plsc.addupdate_scatter(acc, [idx], v * wv, mask=mv)
                return 0

            lax.fori_loop(0, BLKPIX // 16, chunk, 0)
            return 0

        lax.fori_loop(0, NBLK, blkbody, 0)
        pltpu.sync_copy(acc, out_hbm.at[wid, pl.ds(lo, BANDW)])


def _border_body(im_ref, sp_ref, out_ref):
    p = im_ref[0, 0]
    i0 = lax.broadcasted_iota(jnp.int32, (H, W), 0)
    i1 = lax.broadcasted_iota(jnp.int32, (H, W), 1)
    j0 = lax.broadcasted_iota(jnp.int32, (32, W), 0)
    j1 = lax.broadcasted_iota(jnp.int32, (32, W), 1)
    k0 = lax.broadcasted_iota(jnp.int32, (H, 32), 0)
    k1 = lax.broadcasted_iota(jnp.int32, (H, 32), 1)
    a_top = ((j0 + j1 == 20) & (j0 < 20)).astype(p.dtype)
    g_col = (i0 == jnp.abs(i1 - 20)).astype(p.dtype)
    b_rows = ((i1 + 20 == i0) & (i0 >= 20)).astype(p.dtype)
    g_left = ((k0 + k1 == 20) & (k1 < 20)).astype(p.dtype)
    f32 = jnp.float32
    top = jnp.dot(jnp.dot(a_top, p, preferred_element_type=f32),
                  g_col, preferred_element_type=f32)
    bot = jnp.dot(b_rows, jnp.dot(p, g_left, preferred_element_type=f32),
                  preferred_element_type=f32)
    top_full = jnp.concatenate(
        [top, jnp.zeros((H - 32, W), p.dtype)], axis=0)
    bot_full = jnp.concatenate(
        [bot, jnp.zeros((H, W - 32), p.dtype)], axis=1)
    out_ref[0, 0] = sp_ref[0, 0] + top_full + bot_full


def kernel(im0, flow):
    imf = im0.reshape(B * C, H * W)
    flf = flow.reshape(B, H * W * 2)
    splat = _sc_splat(imf, flf).reshape(B, C, H, W)
    spec = pl.BlockSpec((1, 1, H, W), lambda i, j: (i, j, 0, 0))
    return pl.pallas_call(
        _border_body,
        grid=(B, C),
        in_specs=[spec, spec],
        out_specs=spec,
        out_shape=jax.ShapeDtypeStruct((B, C, H, W), jnp.float32),
    )(im0, splat)


# speedup vs baseline: 1.7476x; 1.7476x over previous
import functools

import jax
import jax.numpy as jnp
from jax import lax
from jax.experimental import pallas as pl
from jax.experimental.pallas import tpu as pltpu
from jax.experimental.pallas import tpu_sc as plsc

B = 2
C = 16
H = 512
W = 512
PADF = 20.0
NBAND = 4
BAND = H // NBAND
BANDW = BAND * W
ROWBLK = 8
BLKPIX = ROWBLK * W
NBLK = H // ROWBLK

_mesh = plsc.VectorSubcoreMesh(core_axis_name="c", subcore_axis_name="s")


@functools.partial(
    pl.kernel,
    out_type=jax.ShapeDtypeStruct((B * C, H * W), jnp.float32),
    mesh=_mesh,
    scratch_types=[
        pltpu.VMEM((BANDW,), jnp.float32),
        pltpu.VMEM((BLKPIX * 2,), jnp.float32),
        pltpu.VMEM((BLKPIX,), jnp.float32),
    ],
    compiler_params=pltpu.CompilerParams(needs_layout_passes=False),
)
def _sc_splat(im_hbm, flow_hbm, out_hbm, acc, flv, val):
    wid = lax.axis_index("s") * 2 + lax.axis_index("c")
    b = wid // C
    lanes = lax.iota(jnp.int32, 16)

    for band in range(NBAND):
        lo_row = band * BAND
        hi_row = lo_row + BAND
        lo = band * BANDW

        def zbody(i, _):
            base = i * 128
            for j in range(8):
                acc[pl.ds(base + j * 16, 16)] = jnp.zeros((16,), jnp.float32)
            return 0

        lax.fori_loop(0, BANDW // 128, zbody, 0)

        def blkbody(blk, _):
            pltpu.sync_copy(
                flow_hbm.at[b, pl.ds(blk * (BLKPIX * 2), BLKPIX * 2)], flv)
            pltpu.sync_copy(im_hbm.at[wid, pl.ds(blk * BLKPIX, BLKPIX)], val)

            def chunk(ci, _):
                off = ci * 16
                gp = blk * BLKPIX + off + lanes
                dx = plsc.load_gather(flv, [off * 2 + lanes * 2])
                dy = plsc.load_gather(flv, [off * 2 + lanes * 2 + 1])
                row = lax.shift_right_logical(gp, 9)
                col = jnp.bitwise_and(gp, W - 1)
                tx = col.astype(jnp.float32) + dx + PADF
                ty = row.astype(jnp.float32) + dy + PADF
                x0 = tx.astype(jnp.int32)
                x0 = jnp.where(x0.astype(jnp.float32) > tx, x0 - 1, x0)
                y0 = ty.astype(jnp.int32)
                y0 = jnp.where(y0.astype(jnp.float32) > ty, y0 - 1, y0)
                fx = tx - x0.astype(jnp.float32)
                fy = ty - y0.astype(jnp.float32)
                gx = 1.0 - fx
                gy = 1.0 - fy
                x1 = x0 + 1
                y1 = y0 + 1
                vx0 = (x0 >= 0) & (x0 < W)
                vx1 = (x1 >= 0) & (x1 < W)
                vy0 = (y0 >= lo_row) & (y0 < hi_row)
                vy1 = (y1 >= lo_row) & (y1 < hi_row)
                ly0 = y0 * W - lo
                ly1 = y1 * W - lo
                v = val[pl.ds(off, 16)]
                for lyv, xv, mv, wv in (
                        (ly0, x0, vx0 & vy0, gx * gy),
                        (ly0, x1, vx1 & vy0, fx * gy),
                        (ly1, x0, vx0 & vy1, gx * fy),
                        (ly1, x1, vx1 & vy1, fx * fy),
                ):
                    idx = jnp.where(mv, lyv + xv, 0)
                    plsc.addupdate_scatter(acc, [idx], v * wv, mask=mv)
                return 0

            lax.fori_loop(0, BLKPIX // 16, chunk, 0)
            return 0

        lax.fori_loop(0, NBLK, blkbody, 0)
        pltpu.sync_copy(acc, out_hbm.at[wid, pl.ds(lo, BANDW)])


def _border_body(im_ref, sp_ref, out_ref):
    p = im_ref[0, 0]
    i0 = lax.broadcasted_iota(jnp.int32, (H, W), 0)
    i1 = lax.broadcasted_iota(jnp.int32, (H, W), 1)
    j0 = lax.broadcasted_iota(jnp.int32, (32, W), 0)
    j1 = lax.broadcasted_iota(jnp.int32, (32, W), 1)
    k0 = lax.broadcasted_iota(jnp.int32, (H, 32), 0)
    k1 = lax.broadcasted_iota(jnp.int32, (H, 32), 1)
    a_top = ((j0 + j1 == 20) & (j0 < 20)).astype(p.dtype)
    g_col = (i0 == jnp.abs(i1 - 20)).astype(p.dtype)
    b_rows = ((i1 + 20 == i0) & (i0 >= 20)).astype(p.dtype)
    g_left = ((k0 + k1 == 20) & (k1 < 20)).astype(p.dtype)
    f32 = jnp.float32
    top = jnp.dot(jnp.dot(a_top, p, preferred_element_type=f32),
                  g_col, preferred_element_type=f32)
    bot = jnp.dot(b_rows, jnp.dot(p, g_left, preferred_element_type=f32),
                  preferred_element_type=f32)
    top_full = jnp.concatenate(
        [top, jnp.zeros((H - 32, W), p.dtype)], axis=0)
    bot_full = jnp.concatenate(
        [bot, jnp.zeros((H, W - 32), p.dtype)], axis=1)
    out_ref[0, 0] = sp_ref[0, 0] + top_full + bot_full


def kernel(im0, flow):
    imf = im0.reshape(B * C, H * W)
    flf = flow.reshape(B, H * W * 2)
    splat = _sc_splat(imf, flf).reshape(B, C, H, W)
    spec = pl.BlockSpec((1, 1, H, W), lambda i, j: (i, j, 0, 0))
    return pl.pallas_call(
        _border_body,
        grid=(B, C),
        in_specs=[spec, spec],
        out_specs=spec,
        out_shape=jax.ShapeDtypeStruct((B, C, H, W), jnp.float32),
    )(im0, splat)


# speedup vs baseline: 2.6980x; 1.5439x over previous
import functools

import jax
import jax.numpy as jnp
from jax import lax
from jax.experimental import pallas as pl
from jax.experimental.pallas import tpu as pltpu
from jax.experimental.pallas import tpu_sc as plsc

B = 2
C = 16
H = 512
W = 512
PADF = 20.0
NBAND = 4
BAND = H // NBAND
BANDW = BAND * W
ROWBLK = 8
BLKPIX = ROWBLK * W
NBLK = H // ROWBLK

_mesh = plsc.VectorSubcoreMesh(core_axis_name="c", subcore_axis_name="s")


@functools.partial(
    pl.kernel,
    out_type=jax.ShapeDtypeStruct((B * C, H * W), jnp.float32),
    mesh=_mesh,
    scratch_types=[
        pltpu.VMEM((BANDW,), jnp.float32),
        pltpu.VMEM((BLKPIX * 2,), jnp.float32),
        pltpu.VMEM((BLKPIX,), jnp.float32),
    ],
    compiler_params=pltpu.CompilerParams(needs_layout_passes=False),
)
def _sc_splat(im_hbm, flow_hbm, out_hbm, acc, flv, val):
    wid = lax.axis_index("s") * 2 + lax.axis_index("c")
    b = wid // C
    lanes = lax.iota(jnp.int32, 16)

    for band in range(NBAND):
        lo_row = band * BAND
        hi_row = lo_row + BAND
        lo = band * BANDW

        def zbody(i, _):
            base = i * 128
            for j in range(8):
                acc[pl.ds(base + j * 16, 16)] = jnp.zeros((16,), jnp.float32)
            return 0

        lax.fori_loop(0, BANDW // 128, zbody, 0)

        def blkbody(blk, _):
            pltpu.sync_copy(
                flow_hbm.at[b, pl.ds(blk * (BLKPIX * 2), BLKPIX * 2)], flv)

            def yscan(ci, carry):
                tmin, tmax = carry
                off = ci * 16
                dy = plsc.load_gather(flv, [off * 2 + lanes * 2 + 1])
                rw = lax.shift_right_logical(blk * BLKPIX + off + lanes, 9)
                ty = rw.astype(jnp.float32) + dy + PADF
                return jnp.minimum(tmin, ty), jnp.maximum(tmax, ty)

            tmin, tmax = lax.fori_loop(
                0, BLKPIX // 16, yscan,
                (jnp.full((16,), 1e30, jnp.float32),
                 jnp.full((16,), -1e30, jnp.float32)))
            tlo = jnp.min(tmin)
            thi = jnp.max(tmax)
            ylo = tlo.astype(jnp.int32)
            ylo = jnp.where(ylo.astype(jnp.float32) > tlo, ylo - 1, ylo)
            yhi = thi.astype(jnp.int32)
            yhi = jnp.where(yhi.astype(jnp.float32) > thi, yhi - 1, yhi)

            def chunk(ci, _):
                off = ci * 16
                gp = blk * BLKPIX + off + lanes
                dx = plsc.load_gather(flv, [off * 2 + lanes * 2])
                dy = plsc.load_gather(flv, [off * 2 + lanes * 2 + 1])
                row = lax.shift_right_logical(gp, 9)
                col = jnp.bitwise_and(gp, W - 1)
                tx = col.astype(jnp.float32) + dx + PADF
                ty = row.astype(jnp.float32) + dy + PADF
                x0 = tx.astype(jnp.int32)
                x0 = jnp.where(x0.astype(jnp.float32) > tx, x0 - 1, x0)
                y0 = ty.astype(jnp.int32)
                y0 = jnp.where(y0.astype(jnp.float32) > ty, y0 - 1, y0)
                fx = tx - x0.astype(jnp.float32)
                fy = ty - y0.astype(jnp.float32)
                gx = 1.0 - fx
                gy = 1.0 - fy
                x1 = x0 + 1
                y1 = y0 + 1
                vx0 = (x0 >= 0) & (x0 < W)
                vx1 = (x1 >= 0) & (x1 < W)
                vy0 = (y0 >= lo_row) & (y0 < hi_row)
                vy1 = (y1 >= lo_row) & (y1 < hi_row)
                ly0 = y0 * W - lo
                ly1 = y1 * W - lo
                v = val[pl.ds(off, 16)]
                for lyv, xv, mv, wv in (
                        (ly0, x0, vx0 & vy0, gx * gy),
                        (ly0, x1, vx1 & vy0, fx * gy),
                        (ly1, x0, vx0 & vy1, gx * fy),
                        (ly1, x1, vx1 & vy1, fx * fy),
                ):
                    idx = jnp.where(mv, lyv + xv, 0)
                    plsc.addupdate_scatter(acc, [idx], v * wv, mask=mv)
                return 0

            @pl.when((yhi >= lo_row - 1) & (ylo < hi_row))
            def _():
                pltpu.sync_copy(
                    im_hbm.at[wid, pl.ds(blk * BLKPIX, BLKPIX)], val)
                lax.fori_loop(0, BLKPIX // 16, chunk, 0)
            return 0

        lax.fori_loop(0, NBLK, blkbody, 0)
        pltpu.sync_copy(acc, out_hbm.at[wid, pl.ds(lo, BANDW)])


def _border_body(im_ref, sp_ref, out_ref):
    p = im_ref[0, 0]
    i0 = lax.broadcasted_iota(jnp.int32, (H, W), 0)
    i1 = lax.broadcasted_iota(jnp.int32, (H, W), 1)
    j0 = lax.broadcasted_iota(jnp.int32, (32, W), 0)
    j1 = lax.broadcasted_iota(jnp.int32, (32, W), 1)
    k0 = lax.broadcasted_iota(jnp.int32, (H, 32), 0)
    k1 = lax.broadcasted_iota(jnp.int32, (H, 32), 1)
    a_top = ((j0 + j1 == 20) & (j0 < 20)).astype(p.dtype)
    g_col = (i0 == jnp.abs(i1 - 20)).astype(p.dtype)
    b_rows = ((i1 + 20 == i0) & (i0 >= 20)).astype(p.dtype)
    g_left = ((k0 + k1 == 20) & (k1 < 20)).astype(p.dtype)
    f32 = jnp.float32
    top = jnp.dot(jnp.dot(a_top, p, preferred_element_type=f32),
                  g_col, preferred_element_type=f32)
    bot = jnp.dot(b_rows, jnp.dot(p, g_left, preferred_element_type=f32),
                  preferred_element_type=f32)
    top_full = jnp.concatenate(
        [top, jnp.zeros((H - 32, W), p.dtype)], axis=0)
    bot_full = jnp.concatenate(
        [bot, jnp.zeros((H, W - 32), p.dtype)], axis=1)
    out_ref[0, 0] = sp_ref[0, 0] + top_full + bot_full


def kernel(im0, flow):
    imf = im0.reshape(B * C, H * W)
    flf = flow.reshape(B, H * W * 2)
    splat = _sc_splat(imf, flf).reshape(B, C, H, W)
    spec = pl.BlockSpec((1, 1, H, W), lambda i, j: (i, j, 0, 0))
    return pl.pallas_call(
        _border_body,
        grid=(B, C),
        in_specs=[spec, spec],
        out_specs=spec,
        out_shape=jax.ShapeDtypeStruct((B, C, H, W), jnp.float32),
    )(im0, splat)
